# Initial kernel scaffold; baseline (speedup 1.0000x reference)
import functools
import math

import jax
import jax.numpy as jnp
from jax import lax
from jax.experimental import pallas as pl
from jax.experimental.pallas import tpu as pltpu
from jax.experimental.pallas import tpu_sc as plsc

N = 10000
E = 320000
D = 128
H = 4
DH = D // H
FF = 256

NC = 2
NS = 16
NW = NC * NS
EPT = E // NW
CHUNK = 80
NCHUNK = EPT // CHUNK
ROWS_PT = N // NS
DENW = 16

_INV_SQRT_DH = 1.0 / math.sqrt(DH)


def _qkv_body(x_ref, s_ref, b_ref, wq_ref, wkv_ref, q_ref, kv_ref):
    x = x_ref[...]
    m = jnp.mean(x, axis=-1, keepdims=True)
    v = jnp.var(x, axis=-1, keepdims=True)
    h = (x - m) / jnp.sqrt(v + 1e-6) * s_ref[...] + b_ref[...]
    q_ref[...] = jnp.dot(h, wq_ref[...], preferred_element_type=jnp.float32)
    kv_ref[...] = jnp.dot(h, wkv_ref[...], preferred_element_type=jnp.float32)


def _qkv_call(x, s, b, wq, wkv):
    blk = 1000
    return pl.pallas_call(
        _qkv_body,
        grid=(N // blk,),
        in_specs=[
            pl.BlockSpec((blk, D), lambda i: (i, 0)),
            pl.BlockSpec((1, D), lambda i: (0, 0)),
            pl.BlockSpec((1, D), lambda i: (0, 0)),
            pl.BlockSpec((D, D), lambda i: (0, 0)),
            pl.BlockSpec((D, 2 * D), lambda i: (0, 0)),
        ],
        out_specs=[
            pl.BlockSpec((blk, D), lambda i: (i, 0)),
            pl.BlockSpec((blk, 2 * D), lambda i: (i, 0)),
        ],
        out_shape=[
            jax.ShapeDtypeStruct((N, D), jnp.float32),
            jax.ShapeDtypeStruct((N, 2 * D), jnp.float32),
        ],
    )(x, s, b, wq, wkv)


_MESH = plsc.VectorSubcoreMesh(core_axis_name="c", subcore_axis_name="s")


@functools.partial(
    pl.kernel,
    out_type=(
        jax.ShapeDtypeStruct((NC, N, D), jnp.float32),
        jax.ShapeDtypeStruct((NC, N, DENW), jnp.float32),
    ),
    mesh=_MESH,
    scratch_types=[
        pltpu.VMEM((CHUNK,), jnp.int32),
        pltpu.VMEM((CHUNK,), jnp.int32),
        pltpu.VMEM((CHUNK, D), jnp.float32),
        pltpu.VMEM((CHUNK, 2 * D), jnp.float32),
        pltpu.VMEM((CHUNK, D), jnp.float32),
        pltpu.VMEM((CHUNK, DENW), jnp.float32),
        pltpu.VMEM_SHARED((N, D), jnp.float32),
        pltpu.VMEM_SHARED((N, DENW), jnp.float32),
        pltpu.SemaphoreType.DMA,
        pltpu.SemaphoreType.DMA,
    ],
)
def _edge_kernel(q_hbm, kv_hbm, src_hbm, dst_hbm, zn_hbm, zd_hbm,
                 num_out, den_out,
                 src_v, dst_v, q_rows, kv_rows, vout, exbuf,
                 num_sh, den_sh, sem0, sem1):
    c = lax.axis_index("c")
    s = lax.axis_index("s")
    wid = s * NC + c
    r0 = s * ROWS_PT

    pltpu.sync_copy(zn_hbm.at[pl.ds(r0, ROWS_PT)], num_sh.at[pl.ds(r0, ROWS_PT)])
    pltpu.sync_copy(zd_hbm.at[pl.ds(r0, ROWS_PT)], den_sh.at[pl.ds(r0, ROWS_PT)])
    plsc.subcore_barrier()

    li = lax.iota(jnp.int32, 16)
    masks = [jnp.where(li % 4 == h, 1.0, 0.0).astype(jnp.float32)
             for h in range(H)]
    base = wid * EPT

    def chunk_body(i, carry):
        off = base + i * CHUNK
        pltpu.sync_copy(src_hbm.at[pl.ds(off, CHUNK)], src_v)
        pltpu.sync_copy(dst_hbm.at[pl.ds(off, CHUNK)], dst_v)
        cp_q = pltpu.async_copy(q_hbm.at[dst_v], q_rows, sem0)
        cp_kv = pltpu.async_copy(kv_hbm.at[src_v], kv_rows, sem1)
        cp_q.wait()
        cp_kv.wait()

        def edge_body(e, carry2):
            prods = [q_rows[e, pl.ds(16 * j, 16)] * kv_rows[e, pl.ds(16 * j, 16)]
                     for j in range(8)]
            ehb = []
            exrow = jnp.zeros((16,), jnp.float32)
            for h in range(H):
                sh = jnp.sum(prods[2 * h] + prods[2 * h + 1]) * _INV_SQRT_DH
                eh = jnp.exp(jnp.full((16,), sh, jnp.float32))
                ehb.append(eh)
                exrow = exrow + eh * masks[h]
            exbuf[e, :] = exrow
            for j in range(8):
                vj = kv_rows[e, pl.ds(D + 16 * j, 16)]
                vout[e, pl.ds(16 * j, 16)] = vj * ehb[j // 2]
            return carry2

        lax.fori_loop(0, CHUNK, edge_body, 0)
        pltpu.sync_copy(vout, num_sh.at[dst_v], add=True)
        pltpu.sync_copy(exbuf, den_sh.at[dst_v], add=True)
        return carry

    lax.fori_loop(0, NCHUNK, chunk_body, 0)
    plsc.subcore_barrier()

    pltpu.sync_copy(num_sh.at[pl.ds(r0, ROWS_PT)],
                    num_out.at[c, pl.ds(r0, ROWS_PT)])
    pltpu.sync_copy(den_sh.at[pl.ds(r0, ROWS_PT)],
                    den_out.at[c, pl.ds(r0, ROWS_PT)])


def _out_body(x_ref, num_ref, den_ref, r_ref, wo_ref, s_ref, b_ref,
              w1_ref, b1_ref, w2_ref, b2_ref, o_ref):
    num = num_ref[0] + num_ref[1]
    den = den_ref[0] + den_ref[1]
    denb = jnp.dot(den, r_ref[...], preferred_element_type=jnp.float32)
    agg = num / (denb + 1e-9)
    n1 = jnp.dot(agg, wo_ref[...], preferred_element_type=jnp.float32)
    n2 = x_ref[...] + n1
    m = jnp.mean(n2, axis=-1, keepdims=True)
    v = jnp.var(n2, axis=-1, keepdims=True)
    h2 = (n2 - m) / jnp.sqrt(v + 1e-6) * s_ref[...] + b_ref[...]
    t = jax.nn.gelu(jnp.dot(h2, w1_ref[...], preferred_element_type=jnp.float32)
                    + b1_ref[...])
    n3 = jnp.dot(t, w2_ref[...], preferred_element_type=jnp.float32) + b2_ref[...]
    o_ref[...] = n2 + n3


def _out_call(x, num2, den2, r, wo, s, b, w1, b1, w2, b2):
    blk = 1000
    return pl.pallas_call(
        _out_body,
        grid=(N // blk,),
        in_specs=[
            pl.BlockSpec((blk, D), lambda i: (i, 0)),
            pl.BlockSpec((NC, blk, D), lambda i: (0, i, 0)),
            pl.BlockSpec((NC, blk, DENW), lambda i: (0, i, 0)),
            pl.BlockSpec((DENW, D), lambda i: (0, 0)),
            pl.BlockSpec((D, D), lambda i: (0, 0)),
            pl.BlockSpec((1, D), lambda i: (0, 0)),
            pl.BlockSpec((1, D), lambda i: (0, 0)),
            pl.BlockSpec((D, FF), lambda i: (0, 0)),
            pl.BlockSpec((1, FF), lambda i: (0, 0)),
            pl.BlockSpec((FF, D), lambda i: (0, 0)),
            pl.BlockSpec((1, D), lambda i: (0, 0)),
        ],
        out_specs=pl.BlockSpec((blk, D), lambda i: (i, 0)),
        out_shape=jax.ShapeDtypeStruct((N, D), jnp.float32),
    )(x, num2, den2, r, wo, s, b, w1, b1, w2, b2)


def kernel(x, edge_index, ln1_s, ln1_b, ln2_s, ln2_b,
           Wq, Wk, Wv, Wo, W1, b1, W2, b2):
    wkv = jnp.concatenate([Wk, Wv], axis=1)
    q, kv = _qkv_call(x, ln1_s.reshape(1, D), ln1_b.reshape(1, D), Wq, wkv)
    src = edge_index[0]
    dst = edge_index[1]
    zn = jnp.zeros((N, D), jnp.float32)
    zd = jnp.zeros((N, DENW), jnp.float32)
    num2, den2 = _edge_kernel(q, kv, src, dst, zn, zd)
    r = (jnp.arange(DENW)[:, None] == (jnp.arange(D)[None, :] // DH)
         ).astype(jnp.float32)
    return _out_call(x, num2, den2, r, Wo, ln2_s.reshape(1, D),
                     ln2_b.reshape(1, D), W1, b1.reshape(1, FF), W2,
                     b2.reshape(1, D))


# speedup vs baseline: 27.2900x; 27.2900x over previous
import functools
import math

import jax
import jax.numpy as jnp
from jax import lax
from jax.experimental import pallas as pl
from jax.experimental.pallas import tpu as pltpu
from jax.experimental.pallas import tpu_sc as plsc

N = 10000
E = 320000
D = 128
H = 4
DH = D // H
FF = 256

NC = 2
NS = 16
NW = NC * NS
EPT = E // NW
CHUNK = 40
NCHUNK = EPT // CHUNK
NP = 10240
ROWS_PT = NP // NS
W = D + 16

_INV_SQRT_DH = 1.0 / math.sqrt(DH)


def _qkv_body(x_ref, s_ref, b_ref, wq_ref, wkv_ref, q_ref, kv_ref):
    x = x_ref[...]
    m = jnp.mean(x, axis=-1, keepdims=True)
    v = jnp.var(x, axis=-1, keepdims=True)
    h = (x - m) / jnp.sqrt(v + 1e-6) * s_ref[...] + b_ref[...]
    q_ref[...] = jnp.dot(h, wq_ref[...], preferred_element_type=jnp.float32)
    kv_ref[...] = jnp.dot(h, wkv_ref[...], preferred_element_type=jnp.float32)


def _qkv_call(x, s, b, wq, wkv):
    blk = 1000
    return pl.pallas_call(
        _qkv_body,
        grid=(N // blk,),
        in_specs=[
            pl.BlockSpec((blk, D), lambda i: (i, 0)),
            pl.BlockSpec((1, D), lambda i: (0, 0)),
            pl.BlockSpec((1, D), lambda i: (0, 0)),
            pl.BlockSpec((D, D), lambda i: (0, 0)),
            pl.BlockSpec((D, 2 * D), lambda i: (0, 0)),
        ],
        out_specs=[
            pl.BlockSpec((blk, D), lambda i: (i, 0)),
            pl.BlockSpec((blk, 2 * D), lambda i: (i, 0)),
        ],
        out_shape=[
            jax.ShapeDtypeStruct((N, D), jnp.float32),
            jax.ShapeDtypeStruct((N, 2 * D), jnp.float32),
        ],
    )(x, s, b, wq, wkv)


_MESH = plsc.VectorSubcoreMesh(core_axis_name="c", subcore_axis_name="s")


@functools.partial(
    pl.kernel,
    out_type=jax.ShapeDtypeStruct((NC, NP, W), jnp.float32),
    mesh=_MESH,
    scratch_types=[
        pltpu.VMEM((2, CHUNK), jnp.int32),
        pltpu.VMEM((2, CHUNK), jnp.int32),
        pltpu.VMEM((CHUNK, D), jnp.float32),
        pltpu.VMEM((CHUNK, D), jnp.float32),
        pltpu.VMEM((CHUNK, 2 * D), jnp.float32),
        pltpu.VMEM((CHUNK, 2 * D), jnp.float32),
        pltpu.VMEM((CHUNK, W), jnp.float32),
        pltpu.VMEM_SHARED((NP, W), jnp.float32),
        pltpu.SemaphoreType.DMA,
        pltpu.SemaphoreType.DMA,
        pltpu.SemaphoreType.DMA,
        pltpu.SemaphoreType.DMA,
    ],
    compiler_params=pltpu.CompilerParams(use_tc_tiling_on_sc=False),
)
def _edge_kernel(q_hbm, kv_hbm, src_hbm, dst_hbm, zn_hbm, num_out,
                 idx0, idx1, qr0, qr1, kvr0, kvr1, vout, num_sh,
                 sq0, sq1, sk0, sk1):
    c = lax.axis_index("c")
    s = lax.axis_index("s")
    wid = s * NC + c
    r0 = s * ROWS_PT
    base = wid * EPT

    pltpu.sync_copy(zn_hbm.at[pl.ds(r0, ROWS_PT)], num_sh.at[pl.ds(r0, ROWS_PT)])
    plsc.subcore_barrier()

    li = lax.iota(jnp.int32, 16)
    x1, x2, x4, x8 = li ^ 1, li ^ 2, li ^ 4, li ^ 8
    lh = li & 3
    m0, m1, m2 = lh == 0, lh == 1, lh == 2
    mh = li < 4
    hsplat = [jnp.full((16,), h, jnp.int32) for h in range(H)]

    def _shuf(v, idx):
        return lax.gather(
            v, idx[:, None],
            lax.GatherDimensionNumbers(offset_dims=(),
                                       collapsed_slice_dims=(0,),
                                       start_index_map=(0,)),
            slice_sizes=(1,),
            mode=lax.GatherScatterMode.PROMISE_IN_BOUNDS)

    idxb = (idx0, idx1)
    qrb = (qr0, qr1)
    kvrb = (kvr0, kvr1)
    sqb = (sq0, sq1)
    skb = (sk0, sk1)

    def _idx_copies(off, p):
        return (
            (src_hbm.at[pl.ds(off, CHUNK)], idxb[p].at[0]),
            (dst_hbm.at[pl.ds(off, CHUNK)], idxb[p].at[1]),
        )

    def _gathers(p):
        return (
            (q_hbm.at[idxb[p].at[1]], qrb[p], sqb[p]),
            (kv_hbm.at[idxb[p].at[0]], kvrb[p], skb[p]),
        )

    def _scatter(p):
        return (vout, num_sh.at[idxb[p].at[1]])

    def _compute(p):
        q_rows = qrb[p]
        kv_rows = kvrb[p]

        def edge_body(e, carry):
            prods = [q_rows[e, pl.ds(16 * j, 16)] * kv_rows[e, pl.ds(16 * j, 16)]
                     for j in range(8)]
            grp = []
            for h in range(H):
                t = prods[2 * h] + prods[2 * h + 1]
                t = t + _shuf(t, x1)
                t = t + _shuf(t, x2)
                grp.append(t)
            g = jnp.where(m0, grp[0],
                          jnp.where(m1, grp[1], jnp.where(m2, grp[2], grp[3])))
            g = g + _shuf(g, x4)
            g = g + _shuf(g, x8)
            ev = jnp.exp(g * _INV_SQRT_DH)
            ehb = [_shuf(ev, hsplat[h]) for h in range(H)]
            for j in range(8):
                vj = kv_rows[e, pl.ds(D + 16 * j, 16)]
                vout[e, pl.ds(16 * j, 16)] = vj * ehb[j // 2]
            vout[e, pl.ds(D, 16)] = jnp.where(mh, ev, 0.0)
            return carry

        lax.fori_loop(0, CHUNK, edge_body, 0)

    for sc_, dc_ in _idx_copies(base, 0):
        pltpu.sync_copy(sc_, dc_)
    for sc_, dc_, sm_ in _gathers(0):
        pltpu.async_copy(sc_, dc_, sm_)

    def pipe_body(t, carry):
        for b in (0, 1):
            p, nb = b, 1 - b
            j = 2 * t + b
            offn = base + jnp.minimum(j + 1, NCHUNK - 1) * CHUNK

            for sc_, dc_ in _idx_copies(offn, nb):
                pltpu.sync_copy(sc_, dc_)
            for sc_, dc_, sm_ in _gathers(nb):
                pltpu.async_copy(sc_, dc_, sm_)
            for sc_, dc_, sm_ in _gathers(p):
                pltpu.make_async_copy(sc_, dc_, sm_).wait()
            _compute(p)
            ssrc, sdst = _scatter(p)
            pltpu.sync_copy(ssrc, sdst, add=True)
        return carry

    lax.fori_loop(0, NCHUNK // 2, pipe_body, 0)

    for sc_, dc_, sm_ in _gathers(0):
        pltpu.make_async_copy(sc_, dc_, sm_).wait()
    plsc.subcore_barrier()

    pltpu.sync_copy(num_sh.at[pl.ds(r0, ROWS_PT)],
                    num_out.at[c, pl.ds(r0, ROWS_PT)])


def _out_body(x_ref, num_ref, r_ref, wo_ref, s_ref, b_ref,
              w1_ref, b1_ref, w2_ref, b2_ref, o_ref):
    acc = num_ref[0] + num_ref[1]
    num = acc[:, :D]
    den = acc[:, D:]
    denb = jnp.dot(den, r_ref[...], preferred_element_type=jnp.float32)
    agg = num / (denb + 1e-9)
    n1 = jnp.dot(agg, wo_ref[...], preferred_element_type=jnp.float32)
    n2 = x_ref[...] + n1
    m = jnp.mean(n2, axis=-1, keepdims=True)
    v = jnp.var(n2, axis=-1, keepdims=True)
    h2 = (n2 - m) / jnp.sqrt(v + 1e-6) * s_ref[...] + b_ref[...]
    t = jax.nn.gelu(jnp.dot(h2, w1_ref[...], preferred_element_type=jnp.float32)
                    + b1_ref[...])
    n3 = jnp.dot(t, w2_ref[...], preferred_element_type=jnp.float32) + b2_ref[...]
    o_ref[...] = n2 + n3


def _out_call(x, num2, r, wo, s, b, w1, b1, w2, b2):
    blk = 1000
    return pl.pallas_call(
        _out_body,
        grid=(N // blk,),
        in_specs=[
            pl.BlockSpec((blk, D), lambda i: (i, 0)),
            pl.BlockSpec((NC, blk, W), lambda i: (0, i, 0)),
            pl.BlockSpec((16, D), lambda i: (0, 0)),
            pl.BlockSpec((D, D), lambda i: (0, 0)),
            pl.BlockSpec((1, D), lambda i: (0, 0)),
            pl.BlockSpec((1, D), lambda i: (0, 0)),
            pl.BlockSpec((D, FF), lambda i: (0, 0)),
            pl.BlockSpec((1, FF), lambda i: (0, 0)),
            pl.BlockSpec((FF, D), lambda i: (0, 0)),
            pl.BlockSpec((1, D), lambda i: (0, 0)),
        ],
        out_specs=pl.BlockSpec((blk, D), lambda i: (i, 0)),
        out_shape=jax.ShapeDtypeStruct((N, D), jnp.float32),
    )(x, num2, r, wo, s, b, w1, b1, w2, b2)


def kernel(x, edge_index, ln1_s, ln1_b, ln2_s, ln2_b,
           Wq, Wk, Wv, Wo, W1, b1, W2, b2):
    wkv = jnp.concatenate([Wk, Wv], axis=1)
    q, kv = _qkv_call(x, ln1_s.reshape(1, D), ln1_b.reshape(1, D), Wq, wkv)
    src = edge_index[0]
    dst = edge_index[1]
    zn = jnp.zeros((NP, W), jnp.float32)
    num2 = _edge_kernel(q, kv, src, dst, zn)
    r = (jnp.arange(16)[:, None] == (jnp.arange(D)[None, :] // DH)
         ).astype(jnp.float32)
    return _out_call(x, num2, r, Wo, ln2_s.reshape(1, D),
                     ln2_b.reshape(1, D), W1, b1.reshape(1, FF), W2,
                     b2.reshape(1, D))
